# Initial kernel scaffold; baseline (speedup 1.0000x reference)
import jax
import jax.numpy as jnp
from jax.experimental import pallas as pl


def kernel(row_indices, col_indices, row_table, col_table):
    raise NotImplementedError("write your pallas kernel here")


# speedup vs baseline: 6.7456x; 6.7456x over previous
import functools

import jax
import jax.numpy as jnp
from jax import lax
from jax.experimental import pallas as pl
from jax.experimental.pallas import tpu as pltpu
from jax.experimental.pallas import tpu_sc as plsc

_D = 64
_GSZ = 128
_NW = 32
_G = 4


@functools.cache
def _build(n_groups: int, interpret: bool = False):
    per_w = n_groups // _NW
    n_chunks = per_w // _G
    mesh = plsc.VectorSubcoreMesh(core_axis_name="c", subcore_axis_name="s")

    @functools.partial(
        pl.kernel,
        out_type=jax.ShapeDtypeStruct((n_groups, _GSZ, _D), jnp.float32),
        mesh=mesh,
        scratch_types=[
            pltpu.VMEM((_G, _GSZ), jnp.int32),
            pltpu.VMEM((_G, _GSZ), jnp.int32),
            pltpu.VMEM((_G, _GSZ, _D), jnp.float32),
            pltpu.VMEM((_G, _GSZ, _D), jnp.float32),
            pltpu.SemaphoreType.DMA,
        ],
        compiler_params=pltpu.CompilerParams(use_tc_tiling_on_sc=False),
        interpret=interpret,
    )
    def k(row_idx, col_idx, row_tab, col_tab, out, ridx, cidx, rows, cols, sem):
        wid = lax.axis_index("s") * 2 + lax.axis_index("c")
        base = wid * per_w

        @pl.loop(0, n_chunks)
        def _chunk(ci):
            goff = base + ci * _G
            pltpu.sync_copy(row_idx.at[pl.ds(goff, _G)], ridx)
            pltpu.sync_copy(col_idx.at[pl.ds(goff, _G)], cidx)
            descs = []
            for j in range(_G):
                descs.append(
                    pltpu.async_copy(row_tab.at[ridx.at[j]], rows.at[j], sem))
                descs.append(
                    pltpu.async_copy(col_tab.at[cidx.at[j]], cols.at[j], sem))
            for d in descs:
                d.wait()

            for j in range(_G):
                @pl.loop(0, _GSZ)
                def _el(e):
                    for kk in range(_D // 16):
                        sl = pl.ds(kk * 16, 16)
                        rows[j, e, sl] = rows[j, e, sl] + cols[j, e, sl]

            pltpu.sync_copy(rows, out.at[pl.ds(goff, _G)])

    return k


def kernel(row_indices, col_indices, row_table, col_table):
    b, l = row_indices.shape
    n = b * l
    n_groups = n // _GSZ
    ri = row_indices.reshape(n_groups, _GSZ).astype(jnp.int32)
    ci = col_indices.reshape(n_groups, _GSZ).astype(jnp.int32)
    out = _build(n_groups)(ri, ci, row_table, col_table)
    return out.reshape(b, l, _D)
